# baseline (speedup 1.0000x reference)
import numpy as np
import jax
import jax.numpy as jnp
from jax.experimental import pallas as pl

_ALPHA = 4
_CHUNK = 8


def kernel(frames):
    C, T, H, W = frames.shape
    HW = H * W
    Ts = T // _ALPHA
    idx = np.linspace(0, T - 1, Ts).astype(np.int32)

    n_chunks = T // _CHUNK
    sel_local = []
    sel_slow0 = []
    for j in range(n_chunks):
        lo, hi = j * _CHUNK, (j + 1) * _CHUNK
        rows = [int(i - lo) for i in idx if lo <= i < hi]
        first = int(np.searchsorted(idx, lo))
        sel_local.append(rows)
        sel_slow0.append(first)
    per = len(sel_local[0])
    uniform = all(len(r) == per for r in sel_local)
    closed_form = Ts > 1 and np.array_equal(
        idx, (np.arange(Ts) * (T - 1)) // (Ts - 1))

    f = frames.reshape(C, T, HW)

    if uniform and closed_form:
        def body(in_ref, slow_ref, fast_ref):
            j = pl.program_id(1)
            fast_ref[...] = in_ref[...]
            for k in range(per):
                p = per * j + k
                g = (p * (T - 1)) // (Ts - 1)
                l = g - _CHUNK * j
                slow_ref[0, pl.ds(p, 1), :] = in_ref[0, pl.ds(l, 1), :]

        grid = (C, n_chunks)
        slow3, fast3 = pl.pallas_call(
            body,
            grid=grid,
            in_specs=[pl.BlockSpec((1, _CHUNK, HW), lambda c, j: (c, j, 0))],
            out_specs=[
                pl.BlockSpec((1, Ts, HW), lambda c, j: (c, 0, 0)),
                pl.BlockSpec((1, _CHUNK, HW), lambda c, j: (c, j, 0)),
            ],
            out_shape=[
                jax.ShapeDtypeStruct((C, Ts, HW), frames.dtype),
                jax.ShapeDtypeStruct((C, T, HW), frames.dtype),
            ],
        )(f)
    else:
        def body(in_ref, slow_ref, fast_ref):
            fast_ref[...] = in_ref[...]
            for p, i in enumerate(idx):
                slow_ref[0, p, :] = in_ref[0, int(i), :]

        slow3, fast3 = pl.pallas_call(
            body,
            grid=(C,),
            in_specs=[pl.BlockSpec((1, T, HW), lambda c: (c, 0, 0))],
            out_specs=[
                pl.BlockSpec((1, Ts, HW), lambda c: (c, 0, 0)),
                pl.BlockSpec((1, T, HW), lambda c: (c, 0, 0)),
            ],
            out_shape=[
                jax.ShapeDtypeStruct((C, Ts, HW), frames.dtype),
                jax.ShapeDtypeStruct((C, T, HW), frames.dtype),
            ],
        )(f)

    return (slow3.reshape(C, Ts, H, W), fast3.reshape(C, T, H, W))
